# Initial kernel scaffold; baseline (speedup 1.0000x reference)
import jax
import jax.numpy as jnp
from jax.experimental import pallas as pl


def kernel(eeg_nodes, eeg_idx, eeg_attr, Wa1, ba1, Wa2, ba2, Wb1, bb1, Wb2, bb2, Wc1, bc1, Wc2, bc2, Wd1, bd1, Wd2, bd2, Wdem, bdem):
    raise NotImplementedError("write your pallas kernel here")


# speedup vs baseline: 14.4826x; 14.4826x over previous
import functools

import jax
import jax.numpy as jnp
from jax import lax
from jax.experimental import pallas as pl
from jax.experimental.pallas import tpu as pltpu
from jax.experimental.pallas import tpu_sc as plsc

N = 10000
E = 320000
T = 128
H = 512
L = 128

N_PAD = 10240
CH = 128
E_PAD = 327680
NROW = E_PAD // CH
NCH_S = NROW // 32
RPT = N_PAD // 16

BN = 256
GA = N_PAD // BN

_MESH = plsc.VectorSubcoreMesh(
    core_axis_name="c", subcore_axis_name="s", num_cores=2, num_subcores=16
)


IB = 16
NBLK = NCH_S // IB


@functools.partial(
    pl.kernel,
    out_type=jax.ShapeDtypeStruct((2, N_PAD, 128), jnp.float32),
    mesh=_MESH,
    scratch_types=[
        pltpu.VMEM_SHARED((N_PAD, 128), jnp.float32),
        pltpu.VMEM((2, IB, CH), jnp.int32),
        pltpu.VMEM((2, IB, CH), jnp.int32),
        pltpu.VMEM((CH, 128), jnp.float32),
        pltpu.VMEM((CH, 128), jnp.float32),
        pltpu.SemaphoreType.DMA,
        pltpu.SemaphoreType.DMA,
        pltpu.SemaphoreType.DMA,
        pltpu.SemaphoreType.DMA,
        pltpu.SemaphoreType.DMA,
        pltpu.SemaphoreType.DMA,
    ],
)
def _seg128(tbl_h, src_h, dst_h, z_h, out_h,
            acc_s, sidx, didx, rows0, rows1, g0, g1, s0, s1, i0, i1):
    c = lax.axis_index("c")
    s = lax.axis_index("s")
    tid = c * 16 + s
    r0 = s * RPT
    e0 = tid * NCH_S
    isem = (i0, i1)
    pltpu.sync_copy(z_h.at[pl.ds(r0, RPT)], acc_s.at[pl.ds(r0, RPT)])
    pltpu.async_copy(src_h.at[pl.ds(e0, IB)], sidx.at[0], i0)
    pltpu.async_copy(dst_h.at[pl.ds(e0, IB)], didx.at[0], i0)
    plsc.subcore_barrier()

    def pair(args):
        sx, dx, base = args
        cg0 = pltpu.async_copy(tbl_h.at[sx.at[base]], rows0, g0)
        cg1 = pltpu.async_copy(tbl_h.at[sx.at[base + 1]], rows1, g1)
        cg0.wait()
        cs0 = pltpu.async_copy(rows0, acc_s.at[dx.at[base]], s0, add=True)
        cg1.wait()
        cs1 = pltpu.async_copy(rows1, acc_s.at[dx.at[base + 1]], s1, add=True)
        cs0.wait()
        cs1.wait()

    for blk in range(NBLK):
        b = blk % 2
        pltpu.make_async_copy(src_h.at[pl.ds(e0, IB)], sidx.at[b],
                              isem[b]).wait()
        pltpu.make_async_copy(dst_h.at[pl.ds(e0, IB)], didx.at[b],
                              isem[b]).wait()
        if blk + 1 < NBLK:
            nb = (blk + 1) % 2
            off = e0 + (blk + 1) * IB
            pltpu.async_copy(src_h.at[pl.ds(off, IB)], sidx.at[nb], isem[nb])
            pltpu.async_copy(dst_h.at[pl.ds(off, IB)], didx.at[nb], isem[nb])
        lax.fori_loop(
            0, IB // 2,
            lambda i, _, _b=b: (pair((sidx.at[_b], didx.at[_b], 2 * i)), 0)[1],
            0)
    plsc.subcore_barrier()
    pltpu.sync_copy(acc_s.at[pl.ds(r0, RPT)], out_h.at[c, pl.ds(r0, RPT), :])


@functools.partial(
    pl.kernel,
    out_type=jax.ShapeDtypeStruct((2, N_PAD), jnp.float32),
    mesh=_MESH,
    scratch_types=[
        pltpu.VMEM_SHARED((N_PAD,), jnp.float32),
        pltpu.VMEM_SHARED((N_PAD,), jnp.float32),
        pltpu.VMEM((NCH_S, CH), jnp.int32),
        pltpu.VMEM((NCH_S, CH), jnp.int32),
        pltpu.VMEM((CH,), jnp.float32),
        pltpu.SemaphoreType.DMA,
        pltpu.SemaphoreType.DMA,
    ],
)
def _seg1(t_h, src_h, dst_h, z_h, out_h, t_s, acc_s, sidx, didx, vals, g0, s0):
    c = lax.axis_index("c")
    s = lax.axis_index("s")
    tid = c * 16 + s
    r0 = s * RPT
    pltpu.sync_copy(t_h.at[pl.ds(r0, RPT)], t_s.at[pl.ds(r0, RPT)])
    pltpu.sync_copy(z_h.at[pl.ds(r0, RPT)], acc_s.at[pl.ds(r0, RPT)])
    pltpu.sync_copy(src_h.at[pl.ds(tid * NCH_S, NCH_S)], sidx)
    pltpu.sync_copy(dst_h.at[pl.ds(tid * NCH_S, NCH_S)], didx)
    plsc.subcore_barrier()

    def body(j, _):
        pltpu.async_copy(t_s.at[sidx.at[j]], vals, g0).wait()
        pltpu.async_copy(vals, acc_s.at[didx.at[j]], s0, add=True).wait()
        return 0

    lax.fori_loop(0, NCH_S, body, 0)
    plsc.subcore_barrier()
    pltpu.sync_copy(acc_s.at[pl.ds(r0, RPT)], out_h.at[c, pl.ds(r0, RPT)])


def _mlp_a(x, agg, Wa1, ba1, Wa2, ba2, Wb1):
    def body(x_ref, a0_ref, a1_ref, w1, b1, w2, b2, w3, o_ref):
        z = x_ref[...] + a0_ref[0] + a1_ref[0]
        h = jnp.maximum(jnp.dot(z, w1[...], preferred_element_type=jnp.float32)
                        + b1[...], 0.0)
        h = jnp.maximum(jnp.dot(h, w2[...], preferred_element_type=jnp.float32)
                        + b2[...], 0.0)
        o_ref[...] = jnp.dot(h, w3[...], preferred_element_type=jnp.float32)

    return pl.pallas_call(
        body,
        grid=(GA,),
        in_specs=[
            pl.BlockSpec((BN, T), lambda i: (i, 0)),
            pl.BlockSpec((1, BN, T), lambda i: (0, i, 0)),
            pl.BlockSpec((1, BN, T), lambda i: (1, i, 0)),
            pl.BlockSpec((T, H), lambda i: (0, 0)),
            pl.BlockSpec((1, H), lambda i: (0, 0)),
            pl.BlockSpec((H, H), lambda i: (0, 0)),
            pl.BlockSpec((1, H), lambda i: (0, 0)),
            pl.BlockSpec((H, L), lambda i: (0, 0)),
        ],
        out_specs=pl.BlockSpec((BN, L), lambda i: (i, 0)),
        out_shape=jax.ShapeDtypeStruct((N_PAD, L), jnp.float32),
    )(x, agg, agg, Wa1, ba1.reshape(1, H), Wa2, ba2.reshape(1, H), Wb1)


def _mlp_b(p, aggp, bb1, Wb2, bb2):
    def body(p_ref, a0_ref, a1_ref, b1, w2, b2, o_ref):
        q = jnp.maximum(p_ref[...] + a0_ref[0] + a1_ref[0] + b1[...], 0.0)
        o_ref[...] = jnp.dot(q, w2[...], preferred_element_type=jnp.float32) \
            + b2[...]

    return pl.pallas_call(
        body,
        grid=(GA,),
        in_specs=[
            pl.BlockSpec((BN, L), lambda i: (i, 0)),
            pl.BlockSpec((1, BN, L), lambda i: (0, i, 0)),
            pl.BlockSpec((1, BN, L), lambda i: (1, i, 0)),
            pl.BlockSpec((1, L), lambda i: (0, 0)),
            pl.BlockSpec((L, L), lambda i: (0, 0)),
            pl.BlockSpec((1, L), lambda i: (0, 0)),
        ],
        out_specs=pl.BlockSpec((BN, L), lambda i: (i, 0)),
        out_shape=jax.ShapeDtypeStruct((N_PAD, L), jnp.float32),
    )(p, aggp, aggp, bb1.reshape(1, L), Wb2, bb2.reshape(1, L))


def _mlp_c(feat, aggf, Wc1, bc1, Wc2, bc2, Wd1):
    def body(f_ref, a0_ref, a1_ref, w1, b1, w2, b2, wd1, o_ref):
        z = f_ref[...] + a0_ref[0] + a1_ref[0]
        h = jnp.maximum(jnp.dot(z, w1[...], preferred_element_type=jnp.float32)
                        + b1[...], 0.0)
        h = jnp.maximum(jnp.dot(h, w2[...], preferred_element_type=jnp.float32)
                        + b2[...], 0.0)
        o_ref[0, 0, :] = jnp.sum(h * wd1[...], axis=1)

    return pl.pallas_call(
        body,
        grid=(GA,),
        in_specs=[
            pl.BlockSpec((BN, L), lambda i: (i, 0)),
            pl.BlockSpec((1, BN, L), lambda i: (0, i, 0)),
            pl.BlockSpec((1, BN, L), lambda i: (1, i, 0)),
            pl.BlockSpec((L, H), lambda i: (0, 0)),
            pl.BlockSpec((1, H), lambda i: (0, 0)),
            pl.BlockSpec((H, H), lambda i: (0, 0)),
            pl.BlockSpec((1, H), lambda i: (0, 0)),
            pl.BlockSpec((1, H), lambda i: (0, 0)),
        ],
        out_specs=pl.BlockSpec((1, 1, BN), lambda i: (i, 0, 0)),
        out_shape=jax.ShapeDtypeStruct((GA, 1, BN), jnp.float32),
    )(feat, aggf, aggf, Wc1, bc1.reshape(1, H), Wc2, bc2.reshape(1, H),
      Wd1.reshape(1, H))


def _final_d(t3, aggt0, aggt1, feat, wdem, bd1, Wd2, bd2, bdem):
    def body(t_ref, a0_ref, a1_ref, f_ref, wd_ref, bd1_ref, wd2_ref,
             bd2_ref, bdem_ref, reg_ref, dem_ref, acc_ref):
        i = pl.program_id(0)

        @pl.when(i == 0)
        def _():
            acc_ref[...] = jnp.zeros_like(acc_ref)

        acc_ref[...] += jnp.sum(f_ref[...] * wd_ref[...], axis=0,
                                keepdims=True)
        u = t_ref[0, 0, :] + a0_ref[0, 0, :] + a1_ref[0, 0, :] + bd1_ref[0, 0]
        u = jnp.maximum(u, 0.0) * wd2_ref[0, 0] + bd2_ref[0, 0]
        reg_ref[0, 0, :] = jax.nn.sigmoid(u)

        @pl.when(i == GA - 1)
        def _():
            v = jax.nn.sigmoid(jnp.sum(acc_ref[...]) + bdem_ref[0, 0])
            dem_ref[...] = jnp.zeros_like(dem_ref) + v

    smem = pl.BlockSpec(memory_space=pltpu.SMEM)
    return pl.pallas_call(
        body,
        grid=(GA,),
        in_specs=[
            pl.BlockSpec((1, 1, BN), lambda i: (i, 0, 0)),
            pl.BlockSpec((1, 1, BN), lambda i: (i, 0, 0)),
            pl.BlockSpec((1, 1, BN), lambda i: (i, 0, 0)),
            pl.BlockSpec((BN, L), lambda i: (i, 0)),
            pl.BlockSpec((BN, L), lambda i: (i, 0)),
            smem, smem, smem, smem,
        ],
        out_specs=[
            pl.BlockSpec((1, 1, BN), lambda i: (i, 0, 0)),
            pl.BlockSpec((1, 128), lambda i: (0, 0)),
        ],
        out_shape=[
            jax.ShapeDtypeStruct((GA, 1, BN), jnp.float32),
            jax.ShapeDtypeStruct((1, 128), jnp.float32),
        ],
        scratch_shapes=[pltpu.VMEM((1, 128), jnp.float32)],
    )(t3, aggt0, aggt1, feat, wdem, bd1.reshape(1, 1), Wd2.reshape(1, 1),
      bd2.reshape(1, 1), bdem.reshape(1, 1))


def kernel(eeg_nodes, eeg_idx, eeg_attr, Wa1, ba1, Wa2, ba2, Wb1, bb1, Wb2,
           bb2, Wc1, bc1, Wc2, bc2, Wd1, bd1, Wd2, bd2, Wdem, bdem):
    x = jnp.pad(eeg_nodes, ((0, N_PAD - N), (0, 0)))
    pad_ids = N + jnp.arange(E_PAD - E, dtype=jnp.int32) % (N_PAD - N)
    src_p = jnp.concatenate([eeg_idx[0], pad_ids]).reshape(NROW, CH)
    dst_p = jnp.concatenate([eeg_idx[1], pad_ids]).reshape(NROW, CH)
    z128 = jnp.zeros((N_PAD, 128), jnp.float32)
    z1 = jnp.zeros((N_PAD,), jnp.float32)

    agg0 = _seg128(x, src_p, dst_p, z128)
    p = _mlp_a(x, agg0, Wa1, ba1, Wa2, ba2, Wb1)
    aggp = _seg128(p, src_p, dst_p, z128)
    feat = _mlp_b(p, aggp, bb1, Wb2, bb2)
    aggf = _seg128(feat, src_p, dst_p, z128)
    t3 = _mlp_c(feat, aggf, Wc1, bc1, Wc2, bc2, Wd1)
    aggt = _seg1(t3.reshape(N_PAD), src_p, dst_p, z1)
    wdem = jnp.pad(Wdem.reshape(N, L), ((0, N_PAD - N), (0, 0)))
    reg3, dem = _final_d(t3, aggt[0].reshape(GA, 1, BN),
                         aggt[1].reshape(GA, 1, BN), feat, wdem,
                         bd1, Wd2, bd2, bdem)
    region_scores = reg3.reshape(N_PAD)[:N].reshape(N, 1)
    dementia_pred = dem[:, :1]
    return dementia_pred, region_scores


# speedup vs baseline: 15.7299x; 1.0861x over previous
import functools

import jax
import jax.numpy as jnp
from jax import lax
from jax.experimental import pallas as pl
from jax.experimental.pallas import tpu as pltpu
from jax.experimental.pallas import tpu_sc as plsc

N = 10000
E = 320000
T = 128
H = 512
L = 128

N_PAD = 10240
CH = 128
E_PAD = 327680
NROW = E_PAD // CH
NCH_S = NROW // 32
RPT = N_PAD // 16

BN = 256
GA = N_PAD // BN

_MESH = plsc.VectorSubcoreMesh(
    core_axis_name="c", subcore_axis_name="s", num_cores=2, num_subcores=16
)


IB = 16
NBLK = NCH_S // IB


@functools.partial(
    pl.kernel,
    out_type=jax.ShapeDtypeStruct((2, N_PAD, 128), jnp.float32),
    mesh=_MESH,
    scratch_types=[
        pltpu.VMEM_SHARED((N_PAD, 128), jnp.float32),
        pltpu.VMEM((2, IB, CH), jnp.int32),
        pltpu.VMEM((2, IB, CH), jnp.int32),
        pltpu.VMEM((CH, 128), jnp.float32),
        pltpu.VMEM((CH, 128), jnp.float32),
        pltpu.SemaphoreType.DMA,
        pltpu.SemaphoreType.DMA,
        pltpu.SemaphoreType.DMA,
        pltpu.SemaphoreType.DMA,
        pltpu.SemaphoreType.DMA,
        pltpu.SemaphoreType.DMA,
    ],
)
def _seg128(tbl_h, src_h, dst_h, z_h, out_h,
            acc_s, sidx, didx, rows0, rows1, g0, g1, s0, s1, i0, i1):
    c = lax.axis_index("c")
    s = lax.axis_index("s")
    tid = c * 16 + s
    r0 = s * RPT
    e0 = tid * NCH_S
    isem = (i0, i1)
    pltpu.sync_copy(z_h.at[pl.ds(r0, RPT)], acc_s.at[pl.ds(r0, RPT)])
    rows = (rows0, rows1)
    gsem = (g0, g1)
    ssem = (s0, s1)

    def start_idx(blk):
        b = blk % 2
        off = e0 + blk * IB
        pltpu.async_copy(src_h.at[pl.ds(off, IB)], sidx.at[b], isem[b])
        pltpu.async_copy(dst_h.at[pl.ds(off, IB)], didx.at[b], isem[b])

    def wait_idx(blk):
        b = blk % 2
        off = e0 + blk * IB
        pltpu.make_async_copy(src_h.at[pl.ds(off, IB)], sidx.at[b],
                              isem[b]).wait()
        pltpu.make_async_copy(dst_h.at[pl.ds(off, IB)], didx.at[b],
                              isem[b]).wait()

    def start_g(j):
        pltpu.async_copy(tbl_h.at[sidx.at[(j // IB) % 2, j % IB]],
                         rows[j % 2], gsem[j % 2])

    def wait_g(j):
        pltpu.make_async_copy(tbl_h.at[sidx.at[(j // IB) % 2, j % IB]],
                              rows[j % 2], gsem[j % 2]).wait()

    def start_s(j):
        pltpu.async_copy(rows[j % 2],
                         acc_s.at[didx.at[(j // IB) % 2, j % IB]],
                         ssem[j % 2], add=True)

    def wait_s(j):
        pltpu.make_async_copy(rows[j % 2],
                              acc_s.at[didx.at[(j // IB) % 2, j % IB]],
                              ssem[j % 2]).wait()

    start_idx(0)
    start_idx(1)
    plsc.subcore_barrier()
    wait_idx(0)
    start_g(0)
    for j in range(NCH_S):
        wait_g(j)
        if j >= 1:
            wait_s(j - 1)
        if j + 1 < NCH_S:
            if (j + 1) % IB == 0:
                wait_idx((j + 1) // IB)
            start_g(j + 1)
        start_s(j)
        if j % IB == IB - 1 and j // IB + 2 < NBLK:
            start_idx(j // IB + 2)
    wait_s(NCH_S - 1)
    plsc.subcore_barrier()
    pltpu.sync_copy(acc_s.at[pl.ds(r0, RPT)], out_h.at[c, pl.ds(r0, RPT), :])


@functools.partial(
    pl.kernel,
    out_type=jax.ShapeDtypeStruct((2, N_PAD), jnp.float32),
    mesh=_MESH,
    scratch_types=[
        pltpu.VMEM_SHARED((N_PAD,), jnp.float32),
        pltpu.VMEM_SHARED((N_PAD,), jnp.float32),
        pltpu.VMEM((NCH_S, CH), jnp.int32),
        pltpu.VMEM((NCH_S, CH), jnp.int32),
        pltpu.VMEM((CH,), jnp.float32),
        pltpu.SemaphoreType.DMA,
        pltpu.SemaphoreType.DMA,
    ],
)
def _seg1(t_h, src_h, dst_h, z_h, out_h, t_s, acc_s, sidx, didx, vals, g0, s0):
    c = lax.axis_index("c")
    s = lax.axis_index("s")
    tid = c * 16 + s
    r0 = s * RPT
    pltpu.sync_copy(t_h.at[pl.ds(r0, RPT)], t_s.at[pl.ds(r0, RPT)])
    pltpu.sync_copy(z_h.at[pl.ds(r0, RPT)], acc_s.at[pl.ds(r0, RPT)])
    pltpu.sync_copy(src_h.at[pl.ds(tid * NCH_S, NCH_S)], sidx)
    pltpu.sync_copy(dst_h.at[pl.ds(tid * NCH_S, NCH_S)], didx)
    plsc.subcore_barrier()

    def body(j, _):
        pltpu.async_copy(t_s.at[sidx.at[j]], vals, g0).wait()
        pltpu.async_copy(vals, acc_s.at[didx.at[j]], s0, add=True).wait()
        return 0

    lax.fori_loop(0, NCH_S, body, 0)
    plsc.subcore_barrier()
    pltpu.sync_copy(acc_s.at[pl.ds(r0, RPT)], out_h.at[c, pl.ds(r0, RPT)])


def _mlp_a(x, agg, Wa1, ba1, Wa2, ba2, Wb1):
    def body(x_ref, a0_ref, a1_ref, w1, b1, w2, b2, w3, o_ref):
        z = x_ref[...] + a0_ref[0] + a1_ref[0]
        h = jnp.maximum(jnp.dot(z, w1[...], preferred_element_type=jnp.float32)
                        + b1[...], 0.0)
        h = jnp.maximum(jnp.dot(h, w2[...], preferred_element_type=jnp.float32)
                        + b2[...], 0.0)
        o_ref[...] = jnp.dot(h, w3[...], preferred_element_type=jnp.float32)

    return pl.pallas_call(
        body,
        grid=(GA,),
        in_specs=[
            pl.BlockSpec((BN, T), lambda i: (i, 0)),
            pl.BlockSpec((1, BN, T), lambda i: (0, i, 0)),
            pl.BlockSpec((1, BN, T), lambda i: (1, i, 0)),
            pl.BlockSpec((T, H), lambda i: (0, 0)),
            pl.BlockSpec((1, H), lambda i: (0, 0)),
            pl.BlockSpec((H, H), lambda i: (0, 0)),
            pl.BlockSpec((1, H), lambda i: (0, 0)),
            pl.BlockSpec((H, L), lambda i: (0, 0)),
        ],
        out_specs=pl.BlockSpec((BN, L), lambda i: (i, 0)),
        out_shape=jax.ShapeDtypeStruct((N_PAD, L), jnp.float32),
    )(x, agg, agg, Wa1, ba1.reshape(1, H), Wa2, ba2.reshape(1, H), Wb1)


def _mlp_b(p, aggp, bb1, Wb2, bb2):
    def body(p_ref, a0_ref, a1_ref, b1, w2, b2, o_ref):
        q = jnp.maximum(p_ref[...] + a0_ref[0] + a1_ref[0] + b1[...], 0.0)
        o_ref[...] = jnp.dot(q, w2[...], preferred_element_type=jnp.float32) \
            + b2[...]

    return pl.pallas_call(
        body,
        grid=(GA,),
        in_specs=[
            pl.BlockSpec((BN, L), lambda i: (i, 0)),
            pl.BlockSpec((1, BN, L), lambda i: (0, i, 0)),
            pl.BlockSpec((1, BN, L), lambda i: (1, i, 0)),
            pl.BlockSpec((1, L), lambda i: (0, 0)),
            pl.BlockSpec((L, L), lambda i: (0, 0)),
            pl.BlockSpec((1, L), lambda i: (0, 0)),
        ],
        out_specs=pl.BlockSpec((BN, L), lambda i: (i, 0)),
        out_shape=jax.ShapeDtypeStruct((N_PAD, L), jnp.float32),
    )(p, aggp, aggp, bb1.reshape(1, L), Wb2, bb2.reshape(1, L))


def _mlp_c(feat, aggf, Wc1, bc1, Wc2, bc2, Wd1):
    def body(f_ref, a0_ref, a1_ref, w1, b1, w2, b2, wd1, o_ref):
        z = f_ref[...] + a0_ref[0] + a1_ref[0]
        h = jnp.maximum(jnp.dot(z, w1[...], preferred_element_type=jnp.float32)
                        + b1[...], 0.0)
        h = jnp.maximum(jnp.dot(h, w2[...], preferred_element_type=jnp.float32)
                        + b2[...], 0.0)
        o_ref[0, 0, :] = jnp.sum(h * wd1[...], axis=1)

    return pl.pallas_call(
        body,
        grid=(GA,),
        in_specs=[
            pl.BlockSpec((BN, L), lambda i: (i, 0)),
            pl.BlockSpec((1, BN, L), lambda i: (0, i, 0)),
            pl.BlockSpec((1, BN, L), lambda i: (1, i, 0)),
            pl.BlockSpec((L, H), lambda i: (0, 0)),
            pl.BlockSpec((1, H), lambda i: (0, 0)),
            pl.BlockSpec((H, H), lambda i: (0, 0)),
            pl.BlockSpec((1, H), lambda i: (0, 0)),
            pl.BlockSpec((1, H), lambda i: (0, 0)),
        ],
        out_specs=pl.BlockSpec((1, 1, BN), lambda i: (i, 0, 0)),
        out_shape=jax.ShapeDtypeStruct((GA, 1, BN), jnp.float32),
    )(feat, aggf, aggf, Wc1, bc1.reshape(1, H), Wc2, bc2.reshape(1, H),
      Wd1.reshape(1, H))


def _final_d(t3, aggt0, aggt1, feat, wdem, bd1, Wd2, bd2, bdem):
    def body(t_ref, a0_ref, a1_ref, f_ref, wd_ref, bd1_ref, wd2_ref,
             bd2_ref, bdem_ref, reg_ref, dem_ref, acc_ref):
        i = pl.program_id(0)

        @pl.when(i == 0)
        def _():
            acc_ref[...] = jnp.zeros_like(acc_ref)

        acc_ref[...] += jnp.sum(f_ref[...] * wd_ref[...], axis=0,
                                keepdims=True)
        u = t_ref[0, 0, :] + a0_ref[0, 0, :] + a1_ref[0, 0, :] + bd1_ref[0, 0]
        u = jnp.maximum(u, 0.0) * wd2_ref[0, 0] + bd2_ref[0, 0]
        reg_ref[0, 0, :] = jax.nn.sigmoid(u)

        @pl.when(i == GA - 1)
        def _():
            v = jax.nn.sigmoid(jnp.sum(acc_ref[...]) + bdem_ref[0, 0])
            dem_ref[...] = jnp.zeros_like(dem_ref) + v

    smem = pl.BlockSpec(memory_space=pltpu.SMEM)
    return pl.pallas_call(
        body,
        grid=(GA,),
        in_specs=[
            pl.BlockSpec((1, 1, BN), lambda i: (i, 0, 0)),
            pl.BlockSpec((1, 1, BN), lambda i: (i, 0, 0)),
            pl.BlockSpec((1, 1, BN), lambda i: (i, 0, 0)),
            pl.BlockSpec((BN, L), lambda i: (i, 0)),
            pl.BlockSpec((BN, L), lambda i: (i, 0)),
            smem, smem, smem, smem,
        ],
        out_specs=[
            pl.BlockSpec((1, 1, BN), lambda i: (i, 0, 0)),
            pl.BlockSpec((1, 128), lambda i: (0, 0)),
        ],
        out_shape=[
            jax.ShapeDtypeStruct((GA, 1, BN), jnp.float32),
            jax.ShapeDtypeStruct((1, 128), jnp.float32),
        ],
        scratch_shapes=[pltpu.VMEM((1, 128), jnp.float32)],
    )(t3, aggt0, aggt1, feat, wdem, bd1.reshape(1, 1), Wd2.reshape(1, 1),
      bd2.reshape(1, 1), bdem.reshape(1, 1))


def kernel(eeg_nodes, eeg_idx, eeg_attr, Wa1, ba1, Wa2, ba2, Wb1, bb1, Wb2,
           bb2, Wc1, bc1, Wc2, bc2, Wd1, bd1, Wd2, bd2, Wdem, bdem):
    x = jnp.pad(eeg_nodes, ((0, N_PAD - N), (0, 0)))
    pad_ids = N + jnp.arange(E_PAD - E, dtype=jnp.int32) % (N_PAD - N)
    src_p = jnp.concatenate([eeg_idx[0], pad_ids]).reshape(NROW, CH)
    dst_p = jnp.concatenate([eeg_idx[1], pad_ids]).reshape(NROW, CH)
    z128 = jnp.zeros((N_PAD, 128), jnp.float32)
    z1 = jnp.zeros((N_PAD,), jnp.float32)

    agg0 = _seg128(x, src_p, dst_p, z128)
    p = _mlp_a(x, agg0, Wa1, ba1, Wa2, ba2, Wb1)
    aggp = _seg128(p, src_p, dst_p, z128)
    feat = _mlp_b(p, aggp, bb1, Wb2, bb2)
    aggf = _seg128(feat, src_p, dst_p, z128)
    t3 = _mlp_c(feat, aggf, Wc1, bc1, Wc2, bc2, Wd1)
    aggt = _seg1(t3.reshape(N_PAD), src_p, dst_p, z1)
    wdem = jnp.pad(Wdem.reshape(N, L), ((0, N_PAD - N), (0, 0)))
    reg3, dem = _final_d(t3, aggt[0].reshape(GA, 1, BN),
                         aggt[1].reshape(GA, 1, BN), feat, wdem,
                         bd1, Wd2, bd2, bdem)
    region_scores = reg3.reshape(N_PAD)[:N].reshape(N, 1)
    dementia_pred = dem[:, :1]
    return dementia_pred, region_scores
